# Initial kernel scaffold; baseline (speedup 1.0000x reference)
import jax
import jax.numpy as jnp
from jax.experimental import pallas as pl


def kernel(x0, edge_index, edge_param):
    raise NotImplementedError("write your pallas kernel here")


# speedup vs baseline: 22.5083x; 22.5083x over previous
import functools
import jax
import jax.numpy as jnp
from jax import lax
from jax.experimental import pallas as pl
from jax.experimental.pallas import tpu as pltpu
from jax.experimental.pallas import tpu_sc as plsc

N_NODES = 100000
N_EDGES = 3200000
BATCH = 4
VB = 8
N_STEPS = 10
DT = 1.0 / N_STEPS

NPAD = 100096
DUMP = 100088
NW = 16
K = 2000
EPT = N_EDGES // NW
CHUNKS = EPT // K
VREGS = (K * VB) // 16
RI = K // 16
RPT = NPAD // NW


def _tec_body(x0_hbm, src_hbm, des_hbm, w_hbm, out0, out1,
              src_v, des_v, w_v, vs_v, vd_v, cp_v, cn_v, D):
    wid = lax.axis_index("s")
    row_lo = pl.multiple_of(wid * RPT, 8)
    edge_lo = pl.multiple_of(wid * EPT, 8)

    iota = lax.iota(jnp.int32, 16)
    lane_row = lax.shift_right_logical(iota, 3)
    lane_col = lax.bitwise_and(iota, 7)

    def one_step(S_hbm, D_hbm):
        pltpu.sync_copy(S_hbm.at[pl.ds(row_lo, RPT)],
                        D.at[pl.ds(row_lo, RPT)])
        plsc.subcore_barrier()

        def chunk_body(c, carry):
            base = edge_lo + c * K
            pltpu.sync_copy(src_hbm.at[pl.ds(base, K)], src_v)
            pltpu.sync_copy(des_hbm.at[pl.ds(base, K)], des_v)
            pltpu.sync_copy(w_hbm.at[pl.ds(base, K)], w_v)
            pltpu.sync_copy(S_hbm.at[src_v], vs_v)
            pltpu.sync_copy(S_hbm.at[des_v], vd_v)

            def compute(i, carry2):
                rows = i * 2 + lane_row
                g = plsc.load_gather(w_v, [rows])
                vs = plsc.load_gather(vs_v, [rows, lane_col])
                vd = plsc.load_gather(vd_v, [rows, lane_col])
                cc = (g * DT) * (vs - vd)
                plsc.store_scatter(cp_v, [rows, lane_col], cc)
                plsc.store_scatter(cn_v, [rows, lane_col], -cc)
                return carry2

            lax.fori_loop(0, VREGS, compute, 0)

            def remap(i, carry2):
                sl = pl.ds(i * 16, 16)
                sv = src_v[sl]
                src_v[sl] = jnp.where(sv == 0, DUMP, sv)
                dv = des_v[sl]
                des_v[sl] = jnp.where(dv == 0, DUMP, dv)
                return carry2

            lax.fori_loop(0, RI, remap, 0)

            pltpu.sync_copy(cn_v, D.at[src_v], add=True)
            pltpu.sync_copy(cp_v, D.at[des_v], add=True)
            return carry

        lax.fori_loop(0, CHUNKS, chunk_body, 0)
        plsc.subcore_barrier()
        pltpu.sync_copy(D.at[pl.ds(row_lo, RPT)],
                        D_hbm.at[pl.ds(row_lo, RPT)])

    one_step(x0_hbm, out1)

    def double_step(i, carry):
        one_step(out1, out0)
        one_step(out0, out1)
        return carry

    lax.fori_loop(0, (N_STEPS - 2) // 2, double_step, 0)

    one_step(out1, out0)


@jax.jit
def _run(x0p, src, des, w):
    mesh = plsc.VectorSubcoreMesh(core_axis_name="c", subcore_axis_name="s",
                                  num_cores=1)
    kfn = pl.kernel(
        _tec_body,
        mesh=mesh,
        compiler_params=pltpu.CompilerParams(needs_layout_passes=False,
                                             use_tc_tiling_on_sc=False),
        out_type=(jax.ShapeDtypeStruct((NPAD, VB), jnp.float32),
                  jax.ShapeDtypeStruct((NPAD, VB), jnp.float32)),
        scratch_types=[
            pltpu.VMEM((K,), jnp.int32),
            pltpu.VMEM((K,), jnp.int32),
            pltpu.VMEM((K,), jnp.float32),
            pltpu.VMEM((K, VB), jnp.float32),
            pltpu.VMEM((K, VB), jnp.float32),
            pltpu.VMEM((K, VB), jnp.float32),
            pltpu.VMEM((K, VB), jnp.float32),
            pltpu.VMEM_SHARED((NPAD, VB), jnp.float32),
        ],
    )
    return kfn(x0p, src, des, w)


def kernel(x0, edge_index, edge_param):
    src = edge_index[0]
    des = edge_index[1]
    x0p = jnp.zeros((NPAD, VB), jnp.float32)
    x0p = x0p.at[1:N_NODES + 1, :BATCH].set(x0.T)
    out = _run(x0p, src, des, edge_param)
    return out[0][1:N_NODES + 1, :BATCH].T


# speedup vs baseline: 72.7757x; 3.2333x over previous
import functools
import jax
import jax.numpy as jnp
from jax import lax
from jax.experimental import pallas as pl
from jax.experimental.pallas import tpu as pltpu
from jax.experimental.pallas import tpu_sc as plsc

N_NODES = 100000
N_EDGES = 3200000
BATCH = 4
VB = 8
N_STEPS = 10
DT = 1.0 / N_STEPS

NPAD = 100096
DUMP = 100088
NW = 16
K = 1600
EPT = N_EDGES // NW
CHUNKS = EPT // K
VREGS = (K * VB) // 16
RI = K // 16
RPT = NPAD // NW

assert CHUNKS * K == EPT and CHUNKS % 2 == 1 and CHUNKS >= 5


def _tec_body(x0_hbm, src_hbm, des_hbm, w_hbm, out0, out1, *refs):
    setA = refs[0:10]
    setB = refs[10:20]
    D = refs[20]

    wid = lax.axis_index("s")
    row_lo = pl.multiple_of(wid * RPT, 8)
    edge_lo = pl.multiple_of(wid * EPT, 8)

    iota = lax.iota(jnp.int32, 16)
    lane_row = lax.shift_right_logical(iota, 3)
    lane_col = lax.bitwise_and(iota, 7)

    def issue_L(S, c):
        src_v, des_v, w_v, ss_v, ds_v, vs_v, vd_v, sem_l, sem_g, sem_s = S
        base = edge_lo + c * K
        pltpu.async_copy(src_hbm.at[pl.ds(base, K)], src_v, sem_l)
        pltpu.async_copy(des_hbm.at[pl.ds(base, K)], des_v, sem_l)
        pltpu.async_copy(w_hbm.at[pl.ds(base, K)], w_v, sem_l)

    def wait_L(S):
        src_v, des_v, w_v, ss_v, ds_v, vs_v, vd_v, sem_l, sem_g, sem_s = S
        pltpu.make_async_copy(src_hbm.at[pl.ds(edge_lo, K)], src_v, sem_l).wait()
        pltpu.make_async_copy(des_hbm.at[pl.ds(edge_lo, K)], des_v, sem_l).wait()
        pltpu.make_async_copy(w_hbm.at[pl.ds(edge_lo, K)], w_v, sem_l).wait()

    def issue_G(S, S_hbm):
        src_v, des_v, w_v, ss_v, ds_v, vs_v, vd_v, sem_l, sem_g, sem_s = S
        pltpu.async_copy(S_hbm.at[src_v], vs_v, sem_g)
        pltpu.async_copy(S_hbm.at[des_v], vd_v, sem_g)

    def wait_G(S, S_hbm):
        src_v, des_v, w_v, ss_v, ds_v, vs_v, vd_v, sem_l, sem_g, sem_s = S
        pltpu.make_async_copy(S_hbm.at[src_v], vs_v, sem_g).wait()
        pltpu.make_async_copy(S_hbm.at[des_v], vd_v, sem_g).wait()

    def issue_S(S):
        src_v, des_v, w_v, ss_v, ds_v, vs_v, vd_v, sem_l, sem_g, sem_s = S
        pltpu.async_copy(vs_v, D.at[ss_v], sem_s, add=True)
        pltpu.async_copy(vd_v, D.at[ds_v], sem_s, add=True)

    def wait_S(S):
        src_v, des_v, w_v, ss_v, ds_v, vs_v, vd_v, sem_l, sem_g, sem_s = S
        pltpu.make_async_copy(vs_v, D.at[ss_v], sem_s).wait()
        pltpu.make_async_copy(vd_v, D.at[ds_v], sem_s).wait()

    def compute(S):
        src_v, des_v, w_v, ss_v, ds_v, vs_v, vd_v, sem_l, sem_g, sem_s = S

        @plsc.parallel_loop(0, VREGS, unroll=4)
        def _(i):
            rows = i * 2 + lane_row
            g = plsc.load_gather(w_v, [rows])
            vs = plsc.load_gather(vs_v, [rows, lane_col])
            vd = plsc.load_gather(vd_v, [rows, lane_col])
            cc = (g * DT) * (vs - vd)
            plsc.store_scatter(vd_v, [rows, lane_col], cc)
            plsc.store_scatter(vs_v, [rows, lane_col], -cc)

        @plsc.parallel_loop(0, RI, unroll=2)
        def _(i):
            sl = pl.ds(i * 16, 16)
            sv = src_v[sl]
            ss_v[sl] = jnp.where(sv == 0, DUMP, sv)
            dv = des_v[sl]
            ds_v[sl] = jnp.where(dv == 0, DUMP, dv)

    def one_step(S_hbm, D_hbm):
        pltpu.sync_copy(S_hbm.at[pl.ds(row_lo, RPT)],
                        D.at[pl.ds(row_lo, RPT)])
        plsc.subcore_barrier()

        issue_L(setA, 0)
        issue_L(setB, 1)
        wait_L(setA)
        issue_G(setA, S_hbm)

        def body(c, P, Q, first=False, last=False, prefetch=True):
            wait_G(P, S_hbm)
            if not last:
                if not first:
                    wait_S(Q)
                wait_L(Q)
                issue_G(Q, S_hbm)
            compute(P)
            issue_S(P)
            if prefetch:
                issue_L(P, c + 2)

        body(0, setA, setB, first=True)

        def pair(j, carry):
            c = 1 + 2 * j
            body(c, setB, setA)
            body(c + 1, setA, setB)
            return carry

        lax.fori_loop(0, (CHUNKS - 3) // 2, pair, 0)

        body(CHUNKS - 2, setB, setA, prefetch=False)
        body(CHUNKS - 1, setA, setB, last=True, prefetch=False)

        wait_S(setA)
        wait_S(setB)
        plsc.subcore_barrier()
        pltpu.sync_copy(D.at[pl.ds(row_lo, RPT)],
                        D_hbm.at[pl.ds(row_lo, RPT)])

    one_step(x0_hbm, out1)

    def double_step(i, carry):
        one_step(out1, out0)
        one_step(out0, out1)
        return carry

    lax.fori_loop(0, (N_STEPS - 2) // 2, double_step, 0)

    one_step(out1, out0)


@jax.jit
def _run(x0p, src, des, w):
    mesh = plsc.VectorSubcoreMesh(core_axis_name="c", subcore_axis_name="s",
                                  num_cores=1)
    set_types = [
        pltpu.VMEM((K,), jnp.int32),
        pltpu.VMEM((K,), jnp.int32),
        pltpu.VMEM((K,), jnp.float32),
        pltpu.VMEM((K,), jnp.int32),
        pltpu.VMEM((K,), jnp.int32),
        pltpu.VMEM((K, VB), jnp.float32),
        pltpu.VMEM((K, VB), jnp.float32),
        pltpu.SemaphoreType.DMA,
        pltpu.SemaphoreType.DMA,
        pltpu.SemaphoreType.DMA,
    ]
    kfn = pl.kernel(
        _tec_body,
        mesh=mesh,
        compiler_params=pltpu.CompilerParams(needs_layout_passes=False,
                                             use_tc_tiling_on_sc=False),
        out_type=(jax.ShapeDtypeStruct((NPAD, VB), jnp.float32),
                  jax.ShapeDtypeStruct((NPAD, VB), jnp.float32)),
        scratch_types=set_types + set_types
        + [pltpu.VMEM_SHARED((NPAD, VB), jnp.float32)],
    )
    return kfn(x0p, src, des, w)


def kernel(x0, edge_index, edge_param):
    src = edge_index[0]
    des = edge_index[1]
    x0p = jnp.zeros((NPAD, VB), jnp.float32)
    x0p = x0p.at[1:N_NODES + 1, :BATCH].set(x0.T)
    out = _run(x0p, src, des, edge_param)
    return out[0][1:N_NODES + 1, :BATCH].T


# speedup vs baseline: 72.9208x; 1.0020x over previous
import functools
import jax
import jax.numpy as jnp
from jax import lax
from jax.experimental import pallas as pl
from jax.experimental.pallas import tpu as pltpu
from jax.experimental.pallas import tpu_sc as plsc

N_NODES = 100000
N_EDGES = 3200000
BATCH = 4
VB = 8
N_STEPS = 10
DT = 1.0 / N_STEPS

NPAD = 100096
DUMP = 100088
NW = 16
K = 1600
EPT = N_EDGES // NW
CHUNKS = EPT // K
VREGS = (K * BATCH) // 16
RI = K // 16
RPT = NPAD // NW

assert CHUNKS * K == EPT and CHUNKS % 2 == 1 and CHUNKS >= 5


def _tec_body(x0_hbm, src_hbm, des_hbm, w_hbm, out0, out1, *refs):
    setA = refs[0:10]
    setB = refs[10:20]
    D = refs[20]

    wid = lax.axis_index("s")
    row_lo = pl.multiple_of(wid * RPT, 8)
    edge_lo = pl.multiple_of(wid * EPT, 8)

    iota = lax.iota(jnp.int32, 16)
    lane_row = lax.shift_right_logical(iota, 2)
    lane_col = lax.bitwise_and(iota, 3)

    def issue_L(S, c):
        src_v, des_v, w_v, ss_v, ds_v, vs_v, vd_v, sem_l, sem_g, sem_s = S
        base = edge_lo + c * K
        pltpu.async_copy(src_hbm.at[pl.ds(base, K)], src_v, sem_l)
        pltpu.async_copy(des_hbm.at[pl.ds(base, K)], des_v, sem_l)
        pltpu.async_copy(w_hbm.at[pl.ds(base, K)], w_v, sem_l)

    def wait_L(S):
        src_v, des_v, w_v, ss_v, ds_v, vs_v, vd_v, sem_l, sem_g, sem_s = S
        pltpu.make_async_copy(src_hbm.at[pl.ds(edge_lo, K)], src_v, sem_l).wait()
        pltpu.make_async_copy(des_hbm.at[pl.ds(edge_lo, K)], des_v, sem_l).wait()
        pltpu.make_async_copy(w_hbm.at[pl.ds(edge_lo, K)], w_v, sem_l).wait()

    def issue_G(S, S_hbm):
        src_v, des_v, w_v, ss_v, ds_v, vs_v, vd_v, sem_l, sem_g, sem_s = S
        pltpu.async_copy(S_hbm.at[src_v], vs_v, sem_g)
        pltpu.async_copy(S_hbm.at[des_v], vd_v, sem_g)

    def wait_G(S, S_hbm):
        src_v, des_v, w_v, ss_v, ds_v, vs_v, vd_v, sem_l, sem_g, sem_s = S
        pltpu.make_async_copy(S_hbm.at[src_v], vs_v, sem_g).wait()
        pltpu.make_async_copy(S_hbm.at[des_v], vd_v, sem_g).wait()

    def issue_S(S):
        src_v, des_v, w_v, ss_v, ds_v, vs_v, vd_v, sem_l, sem_g, sem_s = S
        pltpu.async_copy(vs_v, D.at[ss_v], sem_s, add=True)
        pltpu.async_copy(vd_v, D.at[ds_v], sem_s, add=True)

    def wait_S(S):
        src_v, des_v, w_v, ss_v, ds_v, vs_v, vd_v, sem_l, sem_g, sem_s = S
        pltpu.make_async_copy(vs_v, D.at[ss_v], sem_s).wait()
        pltpu.make_async_copy(vd_v, D.at[ds_v], sem_s).wait()

    def compute(S):
        src_v, des_v, w_v, ss_v, ds_v, vs_v, vd_v, sem_l, sem_g, sem_s = S

        @plsc.parallel_loop(0, VREGS, unroll=4)
        def _(i):
            rows = i * 4 + lane_row
            g = plsc.load_gather(w_v, [rows])
            vs = plsc.load_gather(vs_v, [rows, lane_col])
            vd = plsc.load_gather(vd_v, [rows, lane_col])
            cc = (g * DT) * (vs - vd)
            plsc.store_scatter(vd_v, [rows, lane_col], cc)
            plsc.store_scatter(vs_v, [rows, lane_col], -cc)

        @plsc.parallel_loop(0, RI, unroll=2)
        def _(i):
            sl = pl.ds(i * 16, 16)
            sv = src_v[sl]
            ss_v[sl] = jnp.where(sv == 0, DUMP, sv)
            dv = des_v[sl]
            ds_v[sl] = jnp.where(dv == 0, DUMP, dv)

    def one_step(S_hbm, D_hbm):
        pltpu.sync_copy(S_hbm.at[pl.ds(row_lo, RPT)],
                        D.at[pl.ds(row_lo, RPT)])
        plsc.subcore_barrier()

        issue_L(setA, 0)
        issue_L(setB, 1)
        wait_L(setA)
        issue_G(setA, S_hbm)

        def body(c, P, Q, first=False, last=False, prefetch=True):
            wait_G(P, S_hbm)
            if not last:
                if not first:
                    wait_S(Q)
                wait_L(Q)
                issue_G(Q, S_hbm)
            compute(P)
            issue_S(P)
            if prefetch:
                issue_L(P, c + 2)

        body(0, setA, setB, first=True)

        def pair(j, carry):
            c = 1 + 2 * j
            body(c, setB, setA)
            body(c + 1, setA, setB)
            return carry

        lax.fori_loop(0, (CHUNKS - 3) // 2, pair, 0)

        body(CHUNKS - 2, setB, setA, prefetch=False)
        body(CHUNKS - 1, setA, setB, last=True, prefetch=False)

        wait_S(setA)
        wait_S(setB)
        plsc.subcore_barrier()
        pltpu.sync_copy(D.at[pl.ds(row_lo, RPT)],
                        D_hbm.at[pl.ds(row_lo, RPT)])

    one_step(x0_hbm, out1)

    def double_step(i, carry):
        one_step(out1, out0)
        one_step(out0, out1)
        return carry

    lax.fori_loop(0, (N_STEPS - 2) // 2, double_step, 0)

    one_step(out1, out0)


@jax.jit
def _run(x0p, src, des, w):
    mesh = plsc.VectorSubcoreMesh(core_axis_name="c", subcore_axis_name="s",
                                  num_cores=1)
    set_types = [
        pltpu.VMEM((K,), jnp.int32),
        pltpu.VMEM((K,), jnp.int32),
        pltpu.VMEM((K,), jnp.float32),
        pltpu.VMEM((K,), jnp.int32),
        pltpu.VMEM((K,), jnp.int32),
        pltpu.VMEM((K, VB), jnp.float32),
        pltpu.VMEM((K, VB), jnp.float32),
        pltpu.SemaphoreType.DMA,
        pltpu.SemaphoreType.DMA,
        pltpu.SemaphoreType.DMA,
    ]
    kfn = pl.kernel(
        _tec_body,
        mesh=mesh,
        compiler_params=pltpu.CompilerParams(needs_layout_passes=False,
                                             use_tc_tiling_on_sc=False),
        out_type=(jax.ShapeDtypeStruct((NPAD, VB), jnp.float32),
                  jax.ShapeDtypeStruct((NPAD, VB), jnp.float32)),
        scratch_types=set_types + set_types
        + [pltpu.VMEM_SHARED((NPAD, VB), jnp.float32)],
    )
    return kfn(x0p, src, des, w)


def kernel(x0, edge_index, edge_param):
    src = edge_index[0]
    des = edge_index[1]
    x0p = jnp.zeros((NPAD, VB), jnp.float32)
    x0p = x0p.at[1:N_NODES + 1, :BATCH].set(x0.T)
    out = _run(x0p, src, des, edge_param)
    return out[0][1:N_NODES + 1, :BATCH].T
